# Initial kernel scaffold; baseline (speedup 1.0000x reference)
import jax
import jax.numpy as jnp
from jax.experimental import pallas as pl


def kernel(x, table, step):
    raise NotImplementedError("write your pallas kernel here")


# speedup vs baseline: 3.5329x; 3.5329x over previous
import functools

import jax
import jax.numpy as jnp
import numpy as np
from jax import lax
from jax.experimental import pallas as pl
from jax.experimental.pallas import tpu as pltpu
from jax.experimental.pallas import tpu_sc as plsc

L = 16
F = 2
T = 1 << 19
N_PTS = 262144
OUT_D = 3 + L * F

PLS = 1.4472692374403782
RES = [float(np.floor(16 * (PLS ** l))) for l in range(L)]

P2 = -1640531535
P3 = 805459861
MASK = T - 1

NC, NS, LANES = 2, 16, 16
NW = NC * NS
PTS_PER_W = N_PTS // NW
P = 512
NCHUNK = PTS_PER_W // P
NG = P // LANES


@functools.partial(
    pl.kernel,
    out_type=jax.ShapeDtypeStruct((OUT_D, N_PTS), jnp.float32),
    mesh=plsc.VectorSubcoreMesh(core_axis_name="c", subcore_axis_name="s",
                                num_cores=NC, num_subcores=NS),
    scratch_types=[
        pltpu.VMEM((NG * 128,), jnp.int32),
        pltpu.VMEM((NG * 128,), jnp.int32),
        pltpu.VMEM((NG * 128,), jnp.float32),
        pltpu.VMEM((NG * 128,), jnp.float32),
        pltpu.VMEM((NG * 128,), jnp.int32),
        pltpu.VMEM((NG * 128,), jnp.int32),
        pltpu.VMEM((OUT_D, P), jnp.float32),
        pltpu.SemaphoreType.DMA,
        pltpu.SemaphoreType.DMA,
    ],
)
def _hash_encode_sc(xT_hbm, tab_hbm, out_hbm,
                    idx0, idx1, w0, w1, words0, words1, fbuf,
                    sem0, sem1):
    wid = lax.axis_index("s") * NC + lax.axis_index("c")
    idxb = (idx0, idx1)
    wb = (w0, w1)
    wordsb = (words0, words1)
    semb = (sem0, sem1)
    hi_mask = jnp.full((LANES,), -65536, jnp.int32)

    def chunk_body(ck, carry):
        base_pt = wid * PTS_PER_W + ck * P

        pltpu.sync_copy(xT_hbm.at[:, pl.ds(base_pt, P)],
                        fbuf.at[pl.ds(0, 3)])

        def compute_idx_w(l, b):
            res = RES[l]
            lvl_base = l * T
            idx_l, w_l = idxb[b], wb[b]

            def g_body(g, carry):
                s = g * LANES
                xv = fbuf[0, pl.ds(s, LANES)]
                yv = fbuf[1, pl.ds(s, LANES)]
                zv = fbuf[2, pl.ds(s, LANES)]
                px = xv * res
                py = yv * res
                pz = zv * res
                ix = px.astype(jnp.int32)
                iy = py.astype(jnp.int32)
                iz = pz.astype(jnp.int32)
                fx = px - ix.astype(jnp.float32)
                fy = py - iy.astype(jnp.float32)
                fz = pz - iz.astype(jnp.float32)
                hx0, hx1 = ix, ix + 1
                hy0 = iy * P2
                hy1 = hy0 + P2
                hz0 = iz * P3
                hz1 = hz0 + P3
                wx0 = 1.0 - fx
                wy0 = 1.0 - fy
                wz0 = 1.0 - fz
                for c in range(8):
                    hx = hx1 if (c & 1) else hx0
                    hy = hy1 if (c & 2) else hy0
                    hz = hz1 if (c & 4) else hz0
                    wx = fx if (c & 1) else wx0
                    wy = fy if (c & 2) else wy0
                    wz = fz if (c & 4) else wz0
                    hv = ((hx ^ hy ^ hz) & MASK) + lvl_base
                    wv = (wx * wy) * wz
                    idx_l[pl.ds(g * 128 + c * LANES, LANES)] = hv
                    w_l[pl.ds(g * 128 + c * LANES, LANES)] = wv
                return carry

            lax.fori_loop(0, NG, g_body, 0)

        def fire(b):
            idx_l, words_l, sem = idxb[b], wordsb[b], semb[b]

            def g_body(g, carry):
                pltpu.async_copy(
                    tab_hbm.at[idx_l.at[pl.ds(g * 128, 128)]],
                    words_l.at[pl.ds(g * 128, 128)], sem)
                return carry

            lax.fori_loop(0, NG, g_body, 0)

        def drain(b):
            pltpu.make_async_copy(tab_hbm.at[pl.ds(0, NG * 128)],
                                  wordsb[b], semb[b]).wait()

        def accumulate(l, b):
            words_l, w_l = wordsb[b], wb[b]

            def g_body(g, carry):
                f0 = jnp.zeros((LANES,), jnp.float32)
                f1 = jnp.zeros((LANES,), jnp.float32)
                for c in range(8):
                    wv = w_l[pl.ds(g * 128 + c * LANES, LANES)]
                    wd = words_l[pl.ds(g * 128 + c * LANES, LANES)]
                    r0 = lax.bitcast_convert_type(
                        lax.shift_left(wd, 16), jnp.float32)
                    r1 = lax.bitcast_convert_type(
                        lax.bitwise_and(wd, hi_mask), jnp.float32)
                    f0 = f0 + wv * r0
                    f1 = f1 + wv * r1
                fbuf[3 + 2 * l, pl.ds(g * LANES, LANES)] = f0
                fbuf[4 + 2 * l, pl.ds(g * LANES, LANES)] = f1
                return carry

            lax.fori_loop(0, NG, g_body, 0)

        for l in range(L):
            b = l & 1
            compute_idx_w(l, b)
            fire(b)
            if l > 0:
                drain(1 - b)
                accumulate(l - 1, 1 - b)
        drain((L - 1) & 1)
        accumulate(L - 1, (L - 1) & 1)

        pltpu.sync_copy(fbuf, out_hbm.at[:, pl.ds(base_pt, P)])
        return carry

    lax.fori_loop(0, NCHUNK, chunk_body, 0)


def kernel(x, table, step):
    xT = jnp.transpose(x)
    tab = lax.bitcast_convert_type(
        table.astype(jnp.bfloat16).reshape(L * T, F), jnp.int32)
    return jnp.transpose(_hash_encode_sc(xT, tab))
